# baseline (speedup 1.0000x reference)
import jax
import jax.numpy as jnp
import numpy as np
from jax.experimental import pallas as pl
from jax.experimental.pallas import tpu as pltpu

_B, _L, _H, _D = 2, 2048, 16, 64
_BH = _B * _H
_U = 40
_QBLK = 512
_CBLK = 256


def _rotl(x, r):
    return ((x << np.uint32(r)) | (x >> np.uint32(32 - r))).astype(np.uint32)


def _threefry2x32(k1, k2, x1, x2):
    rot = ((13, 15, 26, 6), (17, 29, 16, 24))
    ks0, ks1 = np.uint32(k1), np.uint32(k2)
    ks2 = np.uint32(ks0 ^ ks1 ^ np.uint32(0x1BD11BDA))
    x = [x1.astype(np.uint32) + ks0, x2.astype(np.uint32) + ks1]
    keys = (ks0, ks1, ks2)
    for i in range(5):
        for r in rot[i % 2]:
            x[0] = (x[0] + x[1]).astype(np.uint32)
            x[1] = _rotl(x[1], r) ^ x[0]
        x[0] = (x[0] + keys[(i + 1) % 3]).astype(np.uint32)
        x[1] = (x[1] + keys[(i + 2) % 3] + np.uint32(i + 1)).astype(np.uint32)
    return x


def _iota_hi_lo(n):
    i = np.arange(n, dtype=np.uint64)
    return ((i >> np.uint64(32)).astype(np.uint32),
            (i & np.uint64(0xFFFFFFFF)).astype(np.uint32))


def _sample_indices():
    c1, c2 = _iota_hi_lo(2)
    y1, y2 = _threefry2x32(np.uint32(0), np.uint32(42), c1, c2)
    c1, c2 = _iota_hi_lo(_L * _U)
    b1, b2 = _threefry2x32(y1[1], y2[1], c1, c2)
    lower = b1 ^ b2
    return (lower % np.uint32(_L)).astype(np.int32).reshape(_L, _U)


def _count_matrix_T():
    idx_np = _sample_indices()
    ct = np.zeros((_L, _L), dtype=np.float32)
    np.add.at(ct, (idx_np.reshape(-1), np.repeat(np.arange(_L), _U)), 1.0)
    return ct


_CT_NP = _count_matrix_T()


def _per_head(Q, K, V, ct_ref):
    nblk = _L // _QBLK

    m_parts = []
    for b in range(nblk):
        Qb = Q[b * _QBLK:(b + 1) * _QBLK, :]
        St = jax.lax.dot_general(K, Qb, (((1,), (1,)), ((), ())),
                                 preferred_element_type=jnp.float32)
        Cb = ct_ref[:, b * _QBLK:(b + 1) * _QBLK]
        mx = jnp.max(jnp.where(Cb > 0.0, St, -jnp.inf), axis=0, keepdims=True)
        sm = jnp.sum(St * Cb, axis=0, keepdims=True)
        m_parts.append(mx - sm * (1.0 / _L))
    M = jnp.concatenate(m_parts, axis=1)
    Mc = M.reshape(_L, 1)

    kid = jax.lax.broadcasted_iota(jnp.int32, (_L, 1), 0)
    rank_parts = []
    for b in range(nblk):
        Mrow = M[:, b * _QBLK:(b + 1) * _QBLK]
        lid = jax.lax.broadcasted_iota(jnp.int32, (1, _QBLK), 1) + b * _QBLK
        beats = (Mc > Mrow) | ((Mc == Mrow) & (kid < lid))
        rank_parts.append(jnp.sum(jnp.where(beats, 1.0, 0.0),
                                  axis=0, keepdims=True))
    rank = jnp.concatenate(rank_parts, axis=1)
    rankc = rank.reshape(_L, 1)

    iota_u_col = jax.lax.broadcasted_iota(jnp.int32, (_U, 1), 0).astype(jnp.float32)
    G = jnp.where(rank == iota_u_col, 1.0, 0.0)
    iota_u_row = jax.lax.broadcasted_iota(jnp.int32, (1, _U), 1).astype(jnp.float32)
    GT = jnp.where(rankc == iota_u_row, 1.0, 0.0)

    Qr = jnp.dot(G, Q, preferred_element_type=jnp.float32)
    kidf = jax.lax.broadcasted_iota(jnp.int32, (_L, 1), 0).astype(jnp.float32)
    thr = jnp.dot(G, kidf, preferred_element_type=jnp.float32)
    scores = jax.lax.dot_general(Qr, K, (((1,), (1,)), ((), ())),
                                 preferred_element_type=jnp.float32)
    scores = scores * (1.0 / float(np.sqrt(_D)))
    kids = jax.lax.broadcasted_iota(jnp.int32, (_U, _L), 1).astype(jnp.float32)
    masked = jnp.where(kids > thr, -jnp.inf, scores)
    mmax = jnp.max(masked, axis=1, keepdims=True)
    e = jnp.exp(masked - mmax)
    attn = e / jnp.sum(e, axis=1, keepdims=True)
    upd = jnp.dot(attn, V, preferred_element_type=jnp.float32)

    rr = jax.lax.broadcasted_iota(jnp.int32, (_CBLK, _CBLK), 0)
    cc = jax.lax.broadcasted_iota(jnp.int32, (_CBLK, _CBLK), 1)
    tri = jnp.where(rr >= cc, 1.0, 0.0)
    carry = jnp.zeros((1, _D), jnp.float32)
    ctx_parts = []
    for j in range(_L // _CBLK):
        Vb = V[j * _CBLK:(j + 1) * _CBLK, :]
        blk = jnp.dot(tri, Vb, preferred_element_type=jnp.float32) + carry
        ctx_parts.append(blk)
        carry = blk[_CBLK - 1:_CBLK, :]
    ctx = jnp.concatenate(ctx_parts, axis=0)

    scat = jnp.dot(GT, upd, preferred_element_type=jnp.float32)
    return jnp.where(rankc < float(_U), scat, ctx)


def _body(q_hbm, k_hbm, v_hbm, ct_ref, o_ref, q_vm, k_vm, v_vm, sem):
    i = pl.program_id(0)
    b = i // (_H // 2)
    j = i % (_H // 2)

    cps = [
        pltpu.make_async_copy(r.at[b, :, pl.ds(j * 2 * _D, 2 * _D)],
                              dst, sem.at[n])
        for n, (r, dst) in enumerate(
            ((q_hbm, q_vm), (k_hbm, k_vm), (v_hbm, v_vm)))
    ]
    for cp in cps:
        cp.start()
    for cp in cps:
        cp.wait()

    for half in range(2):
        lo = half * _D
        out = _per_head(q_vm[:, lo:lo + _D], k_vm[:, lo:lo + _D],
                        v_vm[:, lo:lo + _D], ct_ref)
        o_ref[half] = out


def kernel(queries, keys, values):
    B, L, H, D = queries.shape
    Q = queries.reshape(B, L, H * D)
    K = keys.reshape(B, L, H * D)
    V = values.reshape(B, L, H * D)
    ct = jnp.asarray(_CT_NP)

    out = pl.pallas_call(
        _body,
        grid=(B * H // 2,),
        in_specs=[
            pl.BlockSpec(memory_space=pl.ANY),
            pl.BlockSpec(memory_space=pl.ANY),
            pl.BlockSpec(memory_space=pl.ANY),
            pl.BlockSpec((L, L), lambda i: (0, 0)),
        ],
        out_specs=pl.BlockSpec((2, L, D), lambda i: (i, 0, 0)),
        out_shape=jax.ShapeDtypeStruct((B * H, L, D), jnp.float32),
        scratch_shapes=[
            pltpu.VMEM((L, 2 * D), jnp.float32),
            pltpu.VMEM((L, 2 * D), jnp.float32),
            pltpu.VMEM((L, 2 * D), jnp.float32),
            pltpu.SemaphoreType.DMA((3,)),
        ],
    )(Q, K, V, ct)

    return out.reshape(B, H, L, D)


# speedup vs baseline: 1.1405x; 1.1405x over previous
import jax
import jax.numpy as jnp
import numpy as np
from jax.experimental import pallas as pl
from jax.experimental.pallas import tpu as pltpu

_B, _L, _H, _D = 2, 2048, 16, 64
_BH = _B * _H
_U = 40
_QBLK = 512
_CBLK = 256


def _rotl(x, r):
    return ((x << np.uint32(r)) | (x >> np.uint32(32 - r))).astype(np.uint32)


def _threefry2x32(k1, k2, x1, x2):
    rot = ((13, 15, 26, 6), (17, 29, 16, 24))
    ks0, ks1 = np.uint32(k1), np.uint32(k2)
    ks2 = np.uint32(ks0 ^ ks1 ^ np.uint32(0x1BD11BDA))
    x = [x1.astype(np.uint32) + ks0, x2.astype(np.uint32) + ks1]
    keys = (ks0, ks1, ks2)
    for i in range(5):
        for r in rot[i % 2]:
            x[0] = (x[0] + x[1]).astype(np.uint32)
            x[1] = _rotl(x[1], r) ^ x[0]
        x[0] = (x[0] + keys[(i + 1) % 3]).astype(np.uint32)
        x[1] = (x[1] + keys[(i + 2) % 3] + np.uint32(i + 1)).astype(np.uint32)
    return x


def _iota_hi_lo(n):
    i = np.arange(n, dtype=np.uint64)
    return ((i >> np.uint64(32)).astype(np.uint32),
            (i & np.uint64(0xFFFFFFFF)).astype(np.uint32))


def _sample_indices():
    c1, c2 = _iota_hi_lo(2)
    y1, y2 = _threefry2x32(np.uint32(0), np.uint32(42), c1, c2)
    c1, c2 = _iota_hi_lo(_L * _U)
    b1, b2 = _threefry2x32(y1[1], y2[1], c1, c2)
    lower = b1 ^ b2
    return (lower % np.uint32(_L)).astype(np.int32).reshape(_L, _U)


def _count_matrix_T():
    idx_np = _sample_indices()
    ct = np.zeros((_L, _L), dtype=np.float32)
    np.add.at(ct, (idx_np.reshape(-1), np.repeat(np.arange(_L), _U)), 1.0)
    return ct


_CT_NP = _count_matrix_T()
_MB_NP = np.where(_CT_NP > 0.0, 0.0, -1e30).astype(np.float32)


def _per_head(Q, K, V, ct_ref, mb_ref):
    nblk = _L // _QBLK

    m_parts = []
    for b in range(nblk):
        Qb = Q[b * _QBLK:(b + 1) * _QBLK, :]
        St = jax.lax.dot_general(K, Qb, (((1,), (1,)), ((), ())),
                                 preferred_element_type=jnp.float32)
        Cb = ct_ref[:, b * _QBLK:(b + 1) * _QBLK]
        Mb = mb_ref[:, b * _QBLK:(b + 1) * _QBLK]
        mx = jnp.max(St + Mb, axis=0, keepdims=True)
        sm = jnp.sum(St * Cb, axis=0, keepdims=True)
        m_parts.append(mx - sm * (1.0 / _L))
    M = jnp.concatenate(m_parts, axis=1)
    Mc = M.reshape(_L, 1)

    kid = jax.lax.broadcasted_iota(jnp.int32, (_L, 1), 0)
    rank_parts = []
    for b in range(nblk):
        Mrow = M[:, b * _QBLK:(b + 1) * _QBLK]
        lid = jax.lax.broadcasted_iota(jnp.int32, (1, _QBLK), 1) + b * _QBLK
        beats = (Mc > Mrow) | ((Mc == Mrow) & (kid < lid))
        rank_parts.append(jnp.sum(jnp.where(beats, 1.0, 0.0),
                                  axis=0, keepdims=True))
    rank = jnp.concatenate(rank_parts, axis=1)
    rankc = rank.reshape(_L, 1)

    iota_u_col = jax.lax.broadcasted_iota(jnp.int32, (_U, 1), 0).astype(jnp.float32)
    G = jnp.where(rank == iota_u_col, 1.0, 0.0)
    iota_u_row = jax.lax.broadcasted_iota(jnp.int32, (1, _U), 1).astype(jnp.float32)
    GT = jnp.where(rankc == iota_u_row, 1.0, 0.0)

    Qr = jnp.dot(G, Q, preferred_element_type=jnp.float32)
    kidf = jax.lax.broadcasted_iota(jnp.int32, (_L, 1), 0).astype(jnp.float32)
    thr = jnp.dot(G, kidf, preferred_element_type=jnp.float32)
    scores = jax.lax.dot_general(Qr, K, (((1,), (1,)), ((), ())),
                                 preferred_element_type=jnp.float32)
    scores = scores * (1.0 / float(np.sqrt(_D)))
    kids = jax.lax.broadcasted_iota(jnp.int32, (_U, _L), 1).astype(jnp.float32)
    masked = jnp.where(kids > thr, -jnp.inf, scores)
    mmax = jnp.max(masked, axis=1, keepdims=True)
    e = jnp.exp(masked - mmax)
    attn = e / jnp.sum(e, axis=1, keepdims=True)
    upd = jnp.dot(attn, V, preferred_element_type=jnp.float32)

    rr = jax.lax.broadcasted_iota(jnp.int32, (_CBLK, _CBLK), 0)
    cc = jax.lax.broadcasted_iota(jnp.int32, (_CBLK, _CBLK), 1)
    tri = jnp.where(rr >= cc, 1.0, 0.0)
    carry = jnp.zeros((1, _D), jnp.float32)
    ctx_parts = []
    for j in range(_L // _CBLK):
        Vb = V[j * _CBLK:(j + 1) * _CBLK, :]
        blk = jnp.dot(tri, Vb, preferred_element_type=jnp.float32) + carry
        ctx_parts.append(blk)
        carry = blk[_CBLK - 1:_CBLK, :]
    ctx = jnp.concatenate(ctx_parts, axis=0)

    scat = jnp.dot(GT, upd, preferred_element_type=jnp.float32)
    return jnp.where(rankc < float(_U), scat, ctx)


def _body(q_hbm, k_hbm, v_hbm, ct_ref, mb_ref, o_ref, q_vm, k_vm, v_vm, sem):
    i = pl.program_id(0)
    nsteps = pl.num_programs(0)

    def start(step, slot):
        b = step // (_H // 2)
        j = step % (_H // 2)
        for n, (r, dst) in enumerate(
                ((q_hbm, q_vm), (k_hbm, k_vm), (v_hbm, v_vm))):
            pltpu.make_async_copy(r.at[b, :, pl.ds(j * 2 * _D, 2 * _D)],
                                  dst.at[slot], sem.at[slot, n]).start()

    def wait(slot):
        for n, (r, dst) in enumerate(
                ((q_hbm, q_vm), (k_hbm, k_vm), (v_hbm, v_vm))):
            pltpu.make_async_copy(r.at[0, :, pl.ds(0, 2 * _D)],
                                  dst.at[slot], sem.at[slot, n]).wait()

    slot = jax.lax.rem(i, 2)
    nslot = jax.lax.rem(i + 1, 2)

    @pl.when(i == 0)
    def _():
        start(i, slot)

    @pl.when(i + 1 < nsteps)
    def _():
        start(i + 1, nslot)

    wait(slot)

    for half in range(2):
        lo = half * _D
        out = _per_head(q_vm[slot, :, lo:lo + _D], k_vm[slot, :, lo:lo + _D],
                        v_vm[slot, :, lo:lo + _D], ct_ref, mb_ref)
        o_ref[half] = out


def kernel(queries, keys, values):
    B, L, H, D = queries.shape
    Q = queries.reshape(B, L, H * D)
    K = keys.reshape(B, L, H * D)
    V = values.reshape(B, L, H * D)
    ct = jnp.asarray(_CT_NP)
    mb = jnp.asarray(_MB_NP)

    out = pl.pallas_call(
        _body,
        grid=(B * H // 2,),
        in_specs=[
            pl.BlockSpec(memory_space=pl.ANY),
            pl.BlockSpec(memory_space=pl.ANY),
            pl.BlockSpec(memory_space=pl.ANY),
            pl.BlockSpec((L, L), lambda i: (0, 0)),
            pl.BlockSpec((L, L), lambda i: (0, 0)),
        ],
        out_specs=pl.BlockSpec((2, L, D), lambda i: (i, 0, 0)),
        out_shape=jax.ShapeDtypeStruct((B * H, L, D), jnp.float32),
        scratch_shapes=[
            pltpu.VMEM((2, L, 2 * D), jnp.float32),
            pltpu.VMEM((2, L, 2 * D), jnp.float32),
            pltpu.VMEM((2, L, 2 * D), jnp.float32),
            pltpu.SemaphoreType.DMA((2, 3)),
        ],
    )(Q, K, V, ct, mb)

    return out.reshape(B, H, L, D)
